# baseline (speedup 1.0000x reference)
import jax
import jax.numpy as jnp
from jax.experimental import pallas as pl
from jax.experimental.pallas import tpu as pltpu
from jax.experimental.pallas import tpu_sc as plsc

_WINDOW = 256


def kernel(input_ids, table):
    batch, fields = input_ids.shape
    emb, dim = table.shape
    num_idx = batch * fields
    assert num_idx % _WINDOW == 0
    grid = num_idx // _WINDOW

    table2 = table.reshape(emb // 2, 2 * dim)
    idx_flat = input_ids.reshape(1, num_idx)
    idx_pair = idx_flat >> 1

    mesh = plsc.VectorSubcoreMesh(
        core_axis_name="core", subcore_axis_name="subcore"
    )

    @pl.kernel(
        out_type=jax.ShapeDtypeStruct((num_idx, 2 * dim), table.dtype),
        mesh=mesh,
    )
    def gather_kernel(table_hbm, idx_hbm, out_hbm):
        def body(idx_vmem, out_vmem):
            pltpu.sync_copy(table_hbm.at[idx_vmem.at[0]], out_vmem)

        pltpu.emit_pipeline(
            body,
            grid=(grid,),
            in_specs=[
                pl.BlockSpec((1, _WINDOW), index_map=lambda i: (0, i))
            ],
            out_specs=[
                pl.BlockSpec(
                    (_WINDOW, 2 * dim), index_map=lambda i: (i, 0)
                )
            ],
            core_axis_name=("core", "subcore"),
            dimension_semantics=(pltpu.PARALLEL,),
        )(idx_hbm, out_hbm)

    pairs = gather_kernel(table2, idx_pair)

    odd = (idx_flat[0] & 1).astype(jnp.bool_)[:, None]
    out = jnp.where(odd, pairs[:, dim:], pairs[:, :dim])
    return out.reshape(batch, fields, dim)
